# Initial kernel scaffold; baseline (speedup 1.0000x reference)
import functools

import jax
import jax.numpy as jnp
from jax import lax
from jax.experimental import pallas as pl
from jax.experimental.pallas import tpu as pltpu
from jax.experimental.pallas import tpu_sc as plsc

_K = 8192
_D = 32
_BETA = 0.25
_BN = 256


def _argmin_body(x2_ref, w2_ref, f_ref, w_ref, idx_ref, dsum_ref):
    i = pl.program_id(0)
    f = f_ref[...]
    w = w_ref[...]
    s = lax.dot_general(f, w, (((1,), (1,)), ((), ())),
                        preferred_element_type=jnp.float32)
    t = x2_ref[...] + w2_ref[...]
    dist = t - 2.0 * s
    m = jnp.min(dist, axis=1, keepdims=True)
    kio = lax.broadcasted_iota(jnp.int32, dist.shape, 1)
    idx = jnp.min(jnp.where(dist == m, kio, _K), axis=1, keepdims=True)
    idx_ref[...] = idx

    @pl.when(i == 0)
    def _init():
        dsum_ref[0, 0] = 0.0

    dsum_ref[0, 0] += jnp.sum(m)


def _argmin_call(x2, w2, flat, emb):
    n = flat.shape[0]
    grid = n // _BN
    return pl.pallas_call(
        _argmin_body,
        grid=(grid,),
        in_specs=[
            pl.BlockSpec((_BN, 1), lambda i: (i, 0)),
            pl.BlockSpec((1, _K), lambda i: (0, 0)),
            pl.BlockSpec((_BN, _D), lambda i: (i, 0)),
            pl.BlockSpec((_K, _D), lambda i: (0, 0)),
        ],
        out_specs=[
            pl.BlockSpec((_BN, 1), lambda i: (i, 0)),
            pl.BlockSpec((1, 1), lambda i: (0, 0)),
        ],
        out_shape=[
            jax.ShapeDtypeStruct((n, 1), jnp.int32),
            jax.ShapeDtypeStruct((1, 1), jnp.float32),
        ],
    )(x2, w2, flat, emb)


def _make_gather(n):
    info = plsc.get_sparse_core_info()
    nw = info.num_cores * info.num_subcores
    b_per_w = n // nw
    chunk = 128
    nchunk = b_per_w // chunk
    mesh = plsc.VectorSubcoreMesh(core_axis_name="c", subcore_axis_name="s")

    @functools.partial(
        pl.kernel,
        mesh=mesh,
        out_type=jax.ShapeDtypeStruct((n, _D), jnp.float32),
        scratch_types=[
            pltpu.VMEM((nchunk, chunk), jnp.int32),
            pltpu.VMEM((chunk, _D), jnp.float32),
            pltpu.SemaphoreType.DMA,
        ],
    )
    def gather(table_hbm, idx_hbm, out_hbm, idx_v, rows_v, sem):
        wid = lax.axis_index("s") * info.num_cores + lax.axis_index("c")
        base = wid * b_per_w
        for j in range(nchunk):
            pltpu.sync_copy(idx_hbm.at[pl.ds(base + j * chunk, chunk)],
                            idx_v.at[j])
        for j in range(nchunk):
            pltpu.async_copy(table_hbm.at[idx_v.at[j]], rows_v, sem).wait()
            pltpu.sync_copy(rows_v,
                            out_hbm.at[pl.ds(base + j * chunk, chunk)])

    return gather


def kernel(z, embedding):
    b, c, h, w = z.shape
    z_nhwc = jnp.transpose(z, (0, 2, 3, 1))
    flat = z_nhwc.reshape(-1, _D).astype(jnp.float32)
    emb = embedding.astype(jnp.float32)
    n = flat.shape[0]

    x2 = jnp.sum(flat ** 2, axis=1, keepdims=True)
    w2 = jnp.sum(emb ** 2, axis=1).reshape(1, _K)

    idx2d, dsum = _argmin_call(x2, w2, flat, emb)
    idx = idx2d.reshape(-1)

    z_q_flat = _make_gather(n)(emb, idx)
    z_q = z_q_flat.reshape(b, h, w, c)
    z_q_out = jnp.transpose(z_q, (0, 3, 1, 2))

    loss = (_BETA / (n * _D)) * dsum[0, 0]
    return (z_q_out, idx, loss)


# speedup vs baseline: 1.1893x; 1.1893x over previous
import functools

import jax
import jax.numpy as jnp
from jax import lax
from jax.experimental import pallas as pl
from jax.experimental.pallas import tpu as pltpu
from jax.experimental.pallas import tpu_sc as plsc

_K = 8192
_D = 32
_BETA = 0.25
_BN = 256


def _argmin_body(x2_ref, w2_ref, f_ref, w_ref, idx_ref, dsum_ref):
    i = pl.program_id(0)
    f = f_ref[...]
    w = w_ref[...]
    s = lax.dot_general(f, w, (((1,), (1,)), ((), ())),
                        preferred_element_type=jnp.float32)
    t = x2_ref[...] + w2_ref[...]
    dist = t - 2.0 * s
    h = _K // 2
    d1 = dist[:, :h]
    d2 = dist[:, h:]
    m1 = jnp.min(d1, axis=1, keepdims=True)
    m2 = jnp.min(d2, axis=1, keepdims=True)
    kio = lax.broadcasted_iota(jnp.int32, d1.shape, 1)
    j1 = jnp.min(jnp.where(d1 == m1, kio, _K), axis=1, keepdims=True)
    j2 = jnp.min(jnp.where(d2 == m2, kio + h, _K), axis=1, keepdims=True)
    m1b = m1.astype(jnp.bfloat16).astype(jnp.float32)
    pick2 = m2 < m1b
    idx_ref[...] = jnp.where(pick2, j2, j1)
    mch = jnp.where(pick2, m2, m1)

    @pl.when(i == 0)
    def _init():
        dsum_ref[...] = jnp.zeros((1, 1), jnp.float32)

    dsum_ref[...] += jnp.sum(mch, axis=(0, 1), keepdims=True)


def _argmin_call(x2, w2, flat, emb):
    n = flat.shape[0]
    grid = n // _BN
    return pl.pallas_call(
        _argmin_body,
        grid=(grid,),
        in_specs=[
            pl.BlockSpec((_BN, 1), lambda i: (i, 0)),
            pl.BlockSpec((1, _K), lambda i: (0, 0)),
            pl.BlockSpec((_BN, _D), lambda i: (i, 0)),
            pl.BlockSpec((_K, _D), lambda i: (0, 0)),
        ],
        out_specs=[
            pl.BlockSpec((_BN, 1), lambda i: (i, 0)),
            pl.BlockSpec((1, 1), lambda i: (0, 0)),
        ],
        out_shape=[
            jax.ShapeDtypeStruct((n, 1), jnp.int32),
            jax.ShapeDtypeStruct((1, 1), jnp.float32),
        ],
    )(x2, w2, flat, emb)


def _make_gather(n):
    info = plsc.get_sparse_core_info()
    nw = info.num_cores * info.num_subcores
    b_per_w = n // nw
    chunk = 128
    nchunk = b_per_w // chunk
    mesh = plsc.VectorSubcoreMesh(core_axis_name="c", subcore_axis_name="s")

    @functools.partial(
        pl.kernel,
        mesh=mesh,
        out_type=jax.ShapeDtypeStruct((n, _D), jnp.float32),
        scratch_types=[
            pltpu.VMEM((nchunk, chunk), jnp.int32),
            pltpu.VMEM((chunk, _D), jnp.float32),
            pltpu.SemaphoreType.DMA,
        ],
        compiler_params=pltpu.CompilerParams(use_tc_tiling_on_sc=False),
    )
    def gather(table_hbm, idx_hbm, out_hbm, idx_v, rows_v, sem):
        wid = lax.axis_index("s") * info.num_cores + lax.axis_index("c")
        base = wid * b_per_w
        for j in range(nchunk):
            pltpu.sync_copy(idx_hbm.at[pl.ds(base + j * chunk, chunk)],
                            idx_v.at[j])
        for j in range(nchunk):
            pltpu.async_copy(table_hbm.at[idx_v.at[j]], rows_v, sem).wait()
            pltpu.sync_copy(rows_v,
                            out_hbm.at[pl.ds(base + j * chunk, chunk)])

    return gather


def kernel(z, embedding):
    b, c, h, w = z.shape
    z_nhwc = jnp.transpose(z, (0, 2, 3, 1))
    flat = z_nhwc.reshape(-1, _D).astype(jnp.float32)
    emb = embedding.astype(jnp.float32)
    n = flat.shape[0]

    x2 = jnp.sum(flat ** 2, axis=1, keepdims=True)
    w2 = jnp.sum(emb ** 2, axis=1).reshape(1, _K)

    idx2d, dsum = _argmin_call(x2, w2, flat, emb)
    idx = idx2d.reshape(-1)

    z_q_flat = _make_gather(n)(emb, idx)
    z_q = z_q_flat.reshape(b, h, w, c)
    z_q_out = jnp.transpose(z_q, (0, 3, 1, 2))

    loss = (_BETA / (n * _D)) * dsum[0, 0]
    return (z_q_out, idx, loss)


# speedup vs baseline: 1.3356x; 1.1230x over previous
import functools

import jax
import jax.numpy as jnp
from jax import lax
from jax.experimental import pallas as pl
from jax.experimental.pallas import tpu as pltpu
from jax.experimental.pallas import tpu_sc as plsc

_K = 8192
_D = 32
_BETA = 0.25
_L = 512


def _argmin_body(x2_ref, w2_ref, z_ref, wneg_ref, idx_ref, dsum_ref):
    i = pl.program_id(0)
    zb = z_ref[0]
    wneg = wneg_ref[...]
    sneg = lax.dot_general(wneg, zb, (((1,), (0,)), ((), ())),
                           preferred_element_type=jnp.float32)
    t = x2_ref[0] + w2_ref[...]
    dist = t + sneg
    h = _K // 2
    d1 = dist[:h]
    d2 = dist[h:]
    m1 = jnp.min(d1, axis=0, keepdims=True)
    m2 = jnp.min(d2, axis=0, keepdims=True)
    m1b = m1.astype(jnp.bfloat16).astype(jnp.float32)
    pick2 = m2 < m1b
    mwin = jnp.where(pick2, m2, m1)
    dwin = jnp.where(pick2, d2, d1)
    kio = lax.broadcasted_iota(jnp.int32, dwin.shape, 0)
    j = jnp.min(jnp.where(dwin == mwin, kio, _K), axis=0, keepdims=True)
    idx = jnp.where(pick2, j + h, j)
    idx_ref[...] = idx.reshape(1, 1, _L)

    @pl.when(i == 0)
    def _init():
        dsum_ref[...] = jnp.zeros((1, 1), jnp.float32)

    dsum_ref[...] += jnp.sum(mwin, axis=(0, 1), keepdims=True)


def _argmin_call(x2, w2, z3, wneg):
    b, _, hw = z3.shape
    c = hw // _L
    grid = b * c
    return pl.pallas_call(
        _argmin_body,
        grid=(grid,),
        in_specs=[
            pl.BlockSpec((1, 1, _L), lambda i: (i // c, 0, i % c)),
            pl.BlockSpec((_K, 1), lambda i: (0, 0)),
            pl.BlockSpec((1, _D, _L), lambda i: (i // c, 0, i % c)),
            pl.BlockSpec((_K, _D), lambda i: (0, 0)),
        ],
        out_specs=[
            pl.BlockSpec((1, 1, _L), lambda i: (i // c, 0, i % c)),
            pl.BlockSpec((1, 1), lambda i: (0, 0)),
        ],
        out_shape=[
            jax.ShapeDtypeStruct((b, 1, hw), jnp.int32),
            jax.ShapeDtypeStruct((1, 1), jnp.float32),
        ],
    )(x2, w2, z3, wneg)


def _make_gather(n):
    info = plsc.get_sparse_core_info()
    nw = info.num_cores * info.num_subcores
    b_per_w = n // nw
    chunk = 128
    nchunk = b_per_w // chunk
    mesh = plsc.VectorSubcoreMesh(core_axis_name="c", subcore_axis_name="s")

    @functools.partial(
        pl.kernel,
        mesh=mesh,
        out_type=jax.ShapeDtypeStruct((n, _D), jnp.float32),
        scratch_types=[
            pltpu.VMEM((nchunk, chunk), jnp.int32),
            pltpu.VMEM((chunk, _D), jnp.float32),
            pltpu.SemaphoreType.DMA,
        ],
        compiler_params=pltpu.CompilerParams(use_tc_tiling_on_sc=False),
    )
    def gather(table_hbm, idx_hbm, out_hbm, idx_v, rows_v, sem):
        wid = lax.axis_index("s") * info.num_cores + lax.axis_index("c")
        base = wid * b_per_w
        for j in range(nchunk):
            pltpu.sync_copy(idx_hbm.at[pl.ds(base + j * chunk, chunk)],
                            idx_v.at[j])
        for j in range(nchunk):
            pltpu.async_copy(table_hbm.at[idx_v.at[j]], rows_v, sem).wait()
            pltpu.sync_copy(rows_v,
                            out_hbm.at[pl.ds(base + j * chunk, chunk)])

    return gather


def kernel(z, embedding):
    b, c, h, w = z.shape
    hw = h * w
    n = b * hw
    emb = embedding.astype(jnp.float32)
    z3 = z.astype(jnp.float32).reshape(b, c, hw)

    x2 = jnp.sum(z3 * z3, axis=1, keepdims=True)
    w2 = jnp.sum(emb ** 2, axis=1, keepdims=True)
    wneg = -2.0 * emb

    idx3, dsum = _argmin_call(x2, w2, z3, wneg)
    idx = idx3.reshape(-1)

    z_q_flat = _make_gather(n)(emb, idx)
    z_q = z_q_flat.reshape(b, h, w, c)
    z_q_out = jnp.transpose(z_q, (0, 3, 1, 2))

    loss = (_BETA / (n * _D)) * dsum[0, 0]
    return (z_q_out, idx, loss)


# speedup vs baseline: 1.4320x; 1.0722x over previous
import functools

import jax
import jax.numpy as jnp
from jax import lax
from jax.experimental import pallas as pl
from jax.experimental.pallas import tpu as pltpu
from jax.experimental.pallas import tpu_sc as plsc

_K = 8192
_D = 32
_BETA = 0.25
_L = 512


def _argmin_body(x2_ref, w2_ref, z_ref, wneg_ref, idx_ref, dsum_ref):
    i = pl.program_id(0)
    zb = z_ref[0]
    wneg = wneg_ref[...]
    sneg = lax.dot_general(wneg, zb, (((1,), (0,)), ((), ())),
                           preferred_element_type=jnp.float32)
    t = x2_ref[0] + w2_ref[...]
    h = _K // 2
    q = _K // 4
    dist = t + sneg
    mq = [jnp.min(dist[a * q:(a + 1) * q], axis=0, keepdims=True)
          for a in range(4)]
    m1 = jnp.minimum(mq[0], mq[1])
    m2 = jnp.minimum(mq[2], mq[3])
    m1b = m1.astype(jnp.bfloat16).astype(jnp.float32)
    pick2 = m2 < m1b
    mwin = jnp.where(pick2, m2, m1)
    sel_b = mq[1] < mq[0]
    sel_d = mq[3] < mq[2]
    dlo = jnp.where(sel_b, dist[q:h], dist[:q])
    dhi = jnp.where(sel_d, dist[h + q:], dist[h:h + q])
    dwin = jnp.where(pick2, dhi, dlo)
    kio = lax.broadcasted_iota(jnp.int32, dwin.shape, 0)
    j = jnp.min(jnp.where(dwin == mwin, kio, _K), axis=0, keepdims=True)
    qlo = jnp.where(sel_b, 1, 0)
    qhi = jnp.where(sel_d, 3, 2)
    idx = j + jnp.where(pick2, qhi, qlo) * q
    idx_ref[...] = idx.reshape(1, 1, _L)

    @pl.when(i == 0)
    def _init():
        dsum_ref[...] = jnp.zeros((1, 1), jnp.float32)

    dsum_ref[...] += jnp.sum(mwin, axis=(0, 1), keepdims=True)


def _argmin_call(x2, w2, z3, wneg):
    b, _, hw = z3.shape
    c = hw // _L
    grid = b * c
    return pl.pallas_call(
        _argmin_body,
        grid=(grid,),
        in_specs=[
            pl.BlockSpec((1, 1, _L), lambda i: (i // c, 0, i % c)),
            pl.BlockSpec((_K, 1), lambda i: (0, 0)),
            pl.BlockSpec((1, _D, _L), lambda i: (i // c, 0, i % c)),
            pl.BlockSpec((_K, _D), lambda i: (0, 0)),
        ],
        out_specs=[
            pl.BlockSpec((1, 1, _L), lambda i: (i // c, 0, i % c)),
            pl.BlockSpec((1, 1), lambda i: (0, 0)),
        ],
        out_shape=[
            jax.ShapeDtypeStruct((b, 1, hw), jnp.int32),
            jax.ShapeDtypeStruct((1, 1), jnp.float32),
        ],
    )(x2, w2, z3, wneg)


def _make_gather(n):
    info = plsc.get_sparse_core_info()
    nw = info.num_cores * info.num_subcores
    b_per_w = n // nw
    chunk = 128
    nchunk = b_per_w // chunk
    mesh = plsc.VectorSubcoreMesh(core_axis_name="c", subcore_axis_name="s")

    @functools.partial(
        pl.kernel,
        mesh=mesh,
        out_type=jax.ShapeDtypeStruct((n, _D), jnp.float32),
        scratch_types=[
            pltpu.VMEM((nchunk, chunk), jnp.int32),
            pltpu.VMEM((chunk, _D), jnp.float32),
            pltpu.SemaphoreType.DMA,
        ],
        compiler_params=pltpu.CompilerParams(use_tc_tiling_on_sc=False),
    )
    def gather(table_hbm, idx_hbm, out_hbm, idx_v, rows_v, sem):
        wid = lax.axis_index("s") * info.num_cores + lax.axis_index("c")
        base = wid * b_per_w
        for j in range(nchunk):
            pltpu.sync_copy(idx_hbm.at[pl.ds(base + j * chunk, chunk)],
                            idx_v.at[j])
        for j in range(nchunk):
            pltpu.async_copy(table_hbm.at[idx_v.at[j]], rows_v, sem).wait()
            pltpu.sync_copy(rows_v,
                            out_hbm.at[pl.ds(base + j * chunk, chunk)])

    return gather


def kernel(z, embedding):
    b, c, h, w = z.shape
    hw = h * w
    n = b * hw
    emb = embedding.astype(jnp.float32)
    z3 = z.astype(jnp.float32).reshape(b, c, hw)

    x2 = jnp.sum(z3 * z3, axis=1, keepdims=True)
    w2 = jnp.sum(emb ** 2, axis=1, keepdims=True)
    wneg = -2.0 * emb

    idx3, dsum = _argmin_call(x2, w2, z3, wneg)
    idx = idx3.reshape(-1)

    z_q_flat = _make_gather(n)(emb, idx)
    z_q = z_q_flat.reshape(b, h, w, c)
    z_q_out = jnp.transpose(z_q, (0, 3, 1, 2))

    loss = (_BETA / (n * _D)) * dsum[0, 0]
    return (z_q_out, idx, loss)


# speedup vs baseline: 1.4504x; 1.0129x over previous
import functools

import jax
import jax.numpy as jnp
from jax import lax
from jax.experimental import pallas as pl
from jax.experimental.pallas import tpu as pltpu
from jax.experimental.pallas import tpu_sc as plsc

_K = 8192
_D = 32
_BETA = 0.25
_L = 512


def _argmin_body(x2_ref, w2_ref, z_ref, wneg_ref, idx_ref, dsum_ref):
    i = pl.program_id(0)
    zb = z_ref[0]
    wneg = wneg_ref[...]
    sneg = lax.dot_general(wneg, zb, (((1,), (0,)), ((), ())),
                           preferred_element_type=jnp.float32)
    t = x2_ref[0] + w2_ref[...]
    h = _K // 2
    q = _K // 4
    e = _K // 8
    dist = t + sneg
    de = [dist[a * e:(a + 1) * e] for a in range(8)]
    me = [jnp.min(d, axis=0, keepdims=True) for d in de]
    m1 = jnp.minimum(jnp.minimum(me[0], me[1]), jnp.minimum(me[2], me[3]))
    m2 = jnp.minimum(jnp.minimum(me[4], me[5]), jnp.minimum(me[6], me[7]))
    m1b = m1.astype(jnp.bfloat16).astype(jnp.float32)
    pick2 = m2 < m1b
    mwin = jnp.where(pick2, m2, m1)
    sel = [me[2 * a + 1] < me[2 * a] for a in range(4)]
    dp = [jnp.where(sel[a], de[2 * a + 1], de[2 * a]) for a in range(4)]
    mp = [jnp.minimum(me[2 * a], me[2 * a + 1]) for a in range(4)]
    sel2 = [mp[1] < mp[0], mp[3] < mp[2]]
    dq = [jnp.where(sel2[0], dp[1], dp[0]),
          jnp.where(sel2[1], dp[3], dp[2])]
    dwin = jnp.where(pick2, dq[1], dq[0])
    kio = lax.broadcasted_iota(jnp.int32, dwin.shape, 0)
    j = jnp.min(jnp.where(dwin == mwin, kio, _K), axis=0, keepdims=True)
    e0 = jnp.where(sel[0], 1, 0)
    e1 = jnp.where(sel[1], 3, 2)
    e2 = jnp.where(sel[2], 5, 4)
    e3 = jnp.where(sel[3], 7, 6)
    elo = jnp.where(sel2[0], e1, e0)
    ehi = jnp.where(sel2[1], e3, e2)
    idx = j + jnp.where(pick2, ehi, elo) * e
    idx_ref[...] = idx.reshape(1, 1, _L)

    @pl.when(i == 0)
    def _init():
        dsum_ref[...] = jnp.zeros((1, 1), jnp.float32)

    dsum_ref[...] += jnp.sum(mwin, axis=(0, 1), keepdims=True)


def _argmin_call(x2, w2, z3, wneg):
    b, _, hw = z3.shape
    c = hw // _L
    grid = b * c
    return pl.pallas_call(
        _argmin_body,
        grid=(grid,),
        in_specs=[
            pl.BlockSpec((1, 1, _L), lambda i: (i // c, 0, i % c)),
            pl.BlockSpec((_K, 1), lambda i: (0, 0)),
            pl.BlockSpec((1, _D, _L), lambda i: (i // c, 0, i % c)),
            pl.BlockSpec((_K, _D), lambda i: (0, 0)),
        ],
        out_specs=[
            pl.BlockSpec((1, 1, _L), lambda i: (i // c, 0, i % c)),
            pl.BlockSpec((1, 1), lambda i: (0, 0)),
        ],
        out_shape=[
            jax.ShapeDtypeStruct((b, 1, hw), jnp.int32),
            jax.ShapeDtypeStruct((1, 1), jnp.float32),
        ],
    )(x2, w2, z3, wneg)


def _make_gather(n):
    info = plsc.get_sparse_core_info()
    nw = info.num_cores * info.num_subcores
    b_per_w = n // nw
    chunk = 128
    nchunk = b_per_w // chunk
    mesh = plsc.VectorSubcoreMesh(core_axis_name="c", subcore_axis_name="s")

    @functools.partial(
        pl.kernel,
        mesh=mesh,
        out_type=jax.ShapeDtypeStruct((n, _D), jnp.float32),
        scratch_types=[
            pltpu.VMEM((nchunk, chunk), jnp.int32),
            pltpu.VMEM((chunk, _D), jnp.float32),
            pltpu.SemaphoreType.DMA,
        ],
        compiler_params=pltpu.CompilerParams(use_tc_tiling_on_sc=False),
    )
    def gather(table_hbm, idx_hbm, out_hbm, idx_v, rows_v, sem):
        wid = lax.axis_index("s") * info.num_cores + lax.axis_index("c")
        base = wid * b_per_w
        for j in range(nchunk):
            pltpu.sync_copy(idx_hbm.at[pl.ds(base + j * chunk, chunk)],
                            idx_v.at[j])
        for j in range(nchunk):
            pltpu.async_copy(table_hbm.at[idx_v.at[j]], rows_v, sem).wait()
            pltpu.sync_copy(rows_v,
                            out_hbm.at[pl.ds(base + j * chunk, chunk)])

    return gather


def kernel(z, embedding):
    b, c, h, w = z.shape
    hw = h * w
    n = b * hw
    emb = embedding.astype(jnp.float32)
    z3 = z.astype(jnp.float32).reshape(b, c, hw)

    x2 = jnp.sum(z3 * z3, axis=1, keepdims=True)
    w2 = jnp.sum(emb ** 2, axis=1, keepdims=True)
    wneg = -2.0 * emb

    idx3, dsum = _argmin_call(x2, w2, z3, wneg)
    idx = idx3.reshape(-1)

    z_q_flat = _make_gather(n)(emb, idx)
    z_q = z_q_flat.reshape(b, h, w, c)
    z_q_out = jnp.transpose(z_q, (0, 3, 1, 2))

    loss = (_BETA / (n * _D)) * dsum[0, 0]
    return (z_q_out, idx, loss)
